# baseline (speedup 1.0000x reference)
import functools
import math

import jax
import jax.numpy as jnp
from jax import lax
from jax.experimental import pallas as pl
from jax.experimental.pallas import tpu as pltpu
from jax.experimental.pallas import tpu_sc as plsc

D_MODEL = 64
SCALE = math.sqrt(D_MODEL)

_info = plsc.get_sparse_core_info()
_NC = _info.num_cores
_NS = _info.num_subcores
_L = _info.num_lanes
_NW = _NC * _NS

TOK = 128
NBUF = 4
NOUT = 4
PREFETCH = 3
BPC = 8


@jax.jit
def kernel(x, lut):
    NB, T = x.shape
    V = lut.shape[0]
    B = NB * T
    n_blocks_total = B // TOK
    blocks_per_w = n_blocks_total // _NW
    bb_per_t = NB // TOK
    n_chunks = blocks_per_w // BPC
    n_pairs = (n_chunks - 1) // 2

    xt = x.T.astype(jnp.int32)
    lut2 = lut.reshape(V // 2, 2 * D_MODEL)

    mesh = plsc.VectorSubcoreMesh(core_axis_name="c", subcore_axis_name="s")

    @functools.partial(
        pl.kernel,
        mesh=mesh,
        out_type=jax.ShapeDtypeStruct((T * D_MODEL, NB), jnp.float32),
        scratch_types=[
            [pltpu.VMEM((BPC * TOK,), jnp.int32) for _ in range(2)],
            [pltpu.VMEM((TOK, 2 * D_MODEL), jnp.float32) for _ in range(NBUF)],
            [pltpu.VMEM((TOK,), jnp.int32) for _ in range(NBUF)],
            [pltpu.VMEM((TOK,), jnp.int32) for _ in range(NBUF)],
            [pltpu.VMEM((D_MODEL, TOK), jnp.float32) for _ in range(NOUT)],
            [pltpu.SemaphoreType.DMA for _ in range(2)],
            [pltpu.SemaphoreType.DMA for _ in range(NBUF)],
            [pltpu.SemaphoreType.DMA for _ in range(NOUT)],
        ],
        compiler_params=pltpu.CompilerParams(
            use_tc_tiling_on_sc=True, needs_layout_passes=False
        ),
    )
    def gather_t(idx_hbm, table_hbm, out_hbm, ichunk, rows, rowids, colbase,
                 blocks, si, sg, ss):
        wid = lax.axis_index("s") * _NC + lax.axis_index("c")
        blk0 = wid * blocks_per_w

        def chunk_slice(c):
            fb = blk0 + c * BPC
            t = lax.shift_right_logical(fb, 5)
            col = jnp.bitwise_and(fb, bb_per_t - 1) * TOK
            return idx_hbm.at[t, pl.ds(col, BPC * TOK)]

        def start_ichunk(c, r):
            pltpu.async_copy(chunk_slice(c), ichunk[r], si[r])

        def wait_ichunk(c, r):
            pltpu.make_async_copy(chunk_slice(c), ichunk[r], si[r]).wait()

        def prep_and_gather(g, j_in_chunk, b, r):
            j8 = j_in_chunk * TOK
            for gg in range(TOK // _L):
                sl = pl.ds(gg * _L, _L)
                v = ichunk[r][pl.ds(j8 + gg * _L, _L)]
                rowids[b][sl] = lax.shift_right_logical(v, 1)
                colbase[b][sl] = lax.shift_left(jnp.bitwise_and(v, 1), 6)
            pltpu.async_copy(table_hbm.at[rowids[b]], rows[b], sg[b])

        def out_slice(g):
            fb = blk0 + g
            t = lax.shift_right_logical(fb, 5)
            bb = jnp.bitwise_and(fb, bb_per_t - 1)
            return out_hbm.at[pl.ds(t * D_MODEL, D_MODEL),
                              pl.ds(bb * TOK, TOK)]

        def wait_store(g_prev, o):
            pltpu.make_async_copy(blocks[o], out_slice(g_prev), ss[o]).wait()

        def transpose_scale(b, o):
            def gg_body(gg, c):
                row_v = lax.iota(jnp.int32, _L) + gg * _L
                cb_v = colbase[b][pl.ds(gg * _L, _L)]
                sl = pl.ds(gg * _L, _L)

                @plsc.parallel_loop(0, D_MODEL, step=1, unroll=8)
                def f_body(f):
                    vals = plsc.load_gather(rows[b], [row_v, cb_v + f])
                    blocks[o][f, sl] = vals * SCALE

                return c

            lax.fori_loop(0, TOK // _L, gg_body, 0)

        def process(g, b, maybe_first):
            pltpu.make_async_copy(table_hbm.at[rowids[b]], rows[b],
                                  sg[b]).wait()
            if maybe_first:
                @pl.when(g >= NOUT)
                def _():
                    wait_store(g - NOUT, b)
            else:
                wait_store(g - NOUT, b)
            transpose_scale(b, b)
            pltpu.async_copy(blocks[b], out_slice(g), ss[b])

        def do_chunk(c, par, first_pair, last):
            g0 = c * BPC
            if not last:
                start_ichunk(c + 1, (par + 1) % 2)
            for j in range(BPC):
                g = g0 + j
                b = j % NBUF
                if not last and j == BPC - PREFETCH:
                    wait_ichunk(c + 1, (par + 1) % 2)
                process(g, b, maybe_first=(first_pair and par == 0))
                if last:
                    if j < BPC - PREFETCH:
                        prep_and_gather(g + PREFETCH, j + PREFETCH,
                                        (b + PREFETCH) % NBUF, par)
                else:
                    h_in_next = j + PREFETCH - BPC
                    if h_in_next < 0:
                        prep_and_gather(g + PREFETCH, j + PREFETCH,
                                        (b + PREFETCH) % NBUF, par)
                    else:
                        prep_and_gather(g + PREFETCH, h_in_next,
                                        (b + PREFETCH) % NBUF, (par + 1) % 2)

        pltpu.async_copy(chunk_slice(0), ichunk[0], si[0])
        wait_ichunk(0, 0)
        for g in range(PREFETCH):
            prep_and_gather(g, g, g % NBUF, 0)

        def pair_body(p, carry):
            c0 = p * 2
            do_chunk(c0, 0, first_pair=True, last=False)
            do_chunk(c0 + 1, 1, first_pair=False, last=False)
            return carry

        lax.fori_loop(0, n_pairs, pair_body, 0)

        do_chunk(n_chunks - 1, 0, first_pair=False, last=True)

        gE = (n_chunks - 1) * BPC
        for j in range(BPC - NOUT, BPC):
            wait_store(gE + j, j % NOUT)

    out2d = gather_t(xt, lut2)
    return out2d.reshape(T, D_MODEL, NB).transpose(2, 0, 1)


# speedup vs baseline: 1.7664x; 1.7664x over previous
import functools
import math

import jax
import jax.numpy as jnp
from jax import lax
from jax.experimental import pallas as pl
from jax.experimental.pallas import tpu as pltpu
from jax.experimental.pallas import tpu_sc as plsc

D_MODEL = 64
SCALE = math.sqrt(D_MODEL)

_info = plsc.get_sparse_core_info()
_NC = _info.num_cores
_NS = _info.num_subcores
_L = _info.num_lanes
_NW = _NC * _NS

TOK = 128
NBUF = 4
NOUT = 4
PREFETCH = 3
BPC = 8


@jax.jit
def kernel(x, lut):
    NB, T = x.shape
    V = lut.shape[0]
    B = NB * T
    n_blocks_total = B // TOK
    blocks_per_w = n_blocks_total // _NW
    bb_per_t = NB // TOK
    n_chunks = blocks_per_w // BPC
    n_pairs = (n_chunks - 1) // 2

    xt = x.T.astype(jnp.int32)
    lut2 = jnp.pad(lut, ((0, 0), (0, 2 * D_MODEL - lut.shape[1])))

    mesh = plsc.VectorSubcoreMesh(core_axis_name="c", subcore_axis_name="s")

    @functools.partial(
        pl.kernel,
        mesh=mesh,
        out_type=jax.ShapeDtypeStruct((T * D_MODEL, NB), jnp.float32),
        scratch_types=[
            [pltpu.VMEM((BPC * TOK,), jnp.int32) for _ in range(2)],
            [pltpu.VMEM((TOK, 2 * D_MODEL), jnp.float32) for _ in range(NBUF)],
            [pltpu.VMEM((D_MODEL, TOK), jnp.float32) for _ in range(NOUT)],
            [pltpu.SemaphoreType.DMA for _ in range(2)],
            [pltpu.SemaphoreType.DMA for _ in range(NBUF)],
            [pltpu.SemaphoreType.DMA for _ in range(NOUT)],
        ],
        compiler_params=pltpu.CompilerParams(
            use_tc_tiling_on_sc=True, needs_layout_passes=False
        ),
    )
    def gather_t(idx_hbm, table_hbm, out_hbm, ichunk, rows, blocks,
                 si, sg, ss):
        wid = lax.axis_index("s") * _NC + lax.axis_index("c")
        blk0 = wid * blocks_per_w

        def chunk_slice(c):
            fb = blk0 + c * BPC
            t = lax.shift_right_logical(fb, 5)
            col = jnp.bitwise_and(fb, bb_per_t - 1) * TOK
            return idx_hbm.at[t, pl.ds(col, BPC * TOK)]

        def start_ichunk(c, r):
            pltpu.async_copy(chunk_slice(c), ichunk[r], si[r])

        def wait_ichunk(c, r):
            pltpu.make_async_copy(chunk_slice(c), ichunk[r], si[r]).wait()

        def idx_list(j_in_chunk, r):
            return ichunk[r].at[pl.ds(j_in_chunk * TOK, TOK)]

        def start_gather(j_in_chunk, b, r):
            pltpu.async_copy(table_hbm.at[idx_list(j_in_chunk, r)],
                             rows[b], sg[b])

        def wait_gather(j_in_chunk, b, r):
            pltpu.make_async_copy(table_hbm.at[idx_list(j_in_chunk, r)],
                                  rows[b], sg[b]).wait()

        def out_slice(g):
            fb = blk0 + g
            t = lax.shift_right_logical(fb, 5)
            bb = jnp.bitwise_and(fb, bb_per_t - 1)
            return out_hbm.at[pl.ds(t * D_MODEL, D_MODEL),
                              pl.ds(bb * TOK, TOK)]

        def wait_store(g_prev, o):
            pltpu.make_async_copy(blocks[o], out_slice(g_prev), ss[o]).wait()

        def transpose_scale(b, o):
            def gg_body(gg, c):
                lane_v = lax.iota(jnp.int32, _L)
                row_v = lane_v + gg * _L

                @plsc.parallel_loop(0, D_MODEL, step=1, unroll=8)
                def f_body(f):
                    fcol = jnp.bitwise_and(lane_v + f, D_MODEL - 1)
                    vals = plsc.load_gather(rows[b], [row_v, fcol])
                    plsc.store_scatter(blocks[o], [fcol, row_v], vals * SCALE)

                return c

            lax.fori_loop(0, TOK // _L, gg_body, 0)

        def process(g, j_in_chunk, b, r, maybe_first):
            wait_gather(j_in_chunk, b, r)
            if maybe_first:
                @pl.when(g >= NOUT)
                def _():
                    wait_store(g - NOUT, b)
            else:
                wait_store(g - NOUT, b)
            transpose_scale(b, b)
            pltpu.async_copy(blocks[b], out_slice(g), ss[b])

        def do_chunk(c, par, first_pair, last):
            g0 = c * BPC
            if not last:
                start_ichunk(c + 1, (par + 1) % 2)
            for j in range(BPC):
                g = g0 + j
                b = j % NBUF
                if not last and j == BPC - PREFETCH:
                    wait_ichunk(c + 1, (par + 1) % 2)
                process(g, j, b, par, maybe_first=(first_pair and par == 0))
                hb = (b + PREFETCH) % NBUF
                if last:
                    if j < BPC - PREFETCH:
                        start_gather(j + PREFETCH, hb, par)
                else:
                    h_in_next = j + PREFETCH - BPC
                    if h_in_next < 0:
                        start_gather(j + PREFETCH, hb, par)
                    else:
                        start_gather(h_in_next, hb, (par + 1) % 2)

        pltpu.async_copy(chunk_slice(0), ichunk[0], si[0])
        wait_ichunk(0, 0)
        for g in range(PREFETCH):
            start_gather(g, g % NBUF, 0)

        def pair_body(p, carry):
            c0 = p * 2
            do_chunk(c0, 0, first_pair=True, last=False)
            do_chunk(c0 + 1, 1, first_pair=False, last=False)
            return carry

        lax.fori_loop(0, n_pairs, pair_body, 0)

        do_chunk(n_chunks - 1, 0, first_pair=False, last=True)

        gE = (n_chunks - 1) * BPC
        for j in range(BPC - NOUT, BPC):
            wait_store(gE + j, j % NOUT)

    out2d = gather_t(xt, lut2)
    return out2d.reshape(T, D_MODEL, NB).transpose(2, 0, 1)
